# Initial kernel scaffold; baseline (speedup 1.0000x reference)
import jax
import jax.numpy as jnp
from jax.experimental import pallas as pl


def kernel(X, transforms_X, transforms_Y):
    raise NotImplementedError("write your pallas kernel here")


# speedup vs baseline: 399.9007x; 399.9007x over previous
import jax
import jax.numpy as jnp
from jax.experimental import pallas as pl

N_ROWS = 1000000
N_COLS = 26
N_SAMPLES = 200
BLOCK_ROWS = 4000
CHUNK_OFF = 72


def _tc_kernel(x_ref, xb0_ref, rinv_ref, ya_ref, yb_ref, da_ref, db_ref, o_ref):
    x = x_ref[...]
    xt = x.T
    xb0 = xb0_ref[...]
    rinv = rinv_ref[...]
    fid = (xt * rinv - xb0 * rinv) * (N_SAMPLES - 1)
    f0 = jnp.floor(fid)
    i0 = jnp.clip(f0.astype(jnp.int32), 0, N_SAMPLES - 2)
    frac = jnp.clip(fid - f0, 0.0, 1.0)

    in_a = i0 < 128
    ia = jnp.minimum(i0, 127)
    ib = jnp.maximum(i0 - CHUNK_OFF, 0)
    y0 = jnp.where(
        in_a,
        jnp.take_along_axis(ya_ref[...], ia, axis=1),
        jnp.take_along_axis(yb_ref[...], ib, axis=1),
    )
    dy = jnp.where(
        in_a,
        jnp.take_along_axis(da_ref[...], ia, axis=1),
        jnp.take_along_axis(db_ref[...], ib, axis=1),
    )
    o_ref[...] = (y0 + frac * dy).T


def kernel(X, transforms_X, transforms_Y):
    xb0 = transforms_X[:, 0].reshape(N_COLS, 1)
    xb1 = transforms_X[:, 1].reshape(N_COLS, 1)
    rinv = 1.0 / (xb1 - xb0)
    dy = transforms_Y[:, 1:] - transforms_Y[:, :-1]
    ya = transforms_Y[:, :128]
    yb = transforms_Y[:, CHUNK_OFF:]
    da = dy[:, :128]
    db = jnp.pad(dy[:, CHUNK_OFF:], ((0, 0), (0, 1)))
    grid = (N_ROWS // BLOCK_ROWS,)
    tbl = lambda i: (0, 0)
    return pl.pallas_call(
        _tc_kernel,
        grid=grid,
        in_specs=[
            pl.BlockSpec((BLOCK_ROWS, N_COLS), lambda i: (i, 0)),
            pl.BlockSpec((N_COLS, 1), tbl),
            pl.BlockSpec((N_COLS, 1), tbl),
            pl.BlockSpec((N_COLS, 128), tbl),
            pl.BlockSpec((N_COLS, 128), tbl),
            pl.BlockSpec((N_COLS, 128), tbl),
            pl.BlockSpec((N_COLS, 128), tbl),
        ],
        out_specs=pl.BlockSpec((BLOCK_ROWS, N_COLS), lambda i: (i, 0)),
        out_shape=jax.ShapeDtypeStruct((N_ROWS, N_COLS), jnp.float32),
    )(X, xb0, rinv, ya, yb, da, db)


# speedup vs baseline: 503.4252x; 1.2589x over previous
import dataclasses
import functools

import jax
import jax.numpy as jnp
from jax.experimental import pallas as pl
from jax.experimental.pallas import tpu as pltpu
from jax.experimental.pallas import tpu_sc as plsc

N_ROWS = 1000000
N_COLS = 26
NS = 200
BR = 1000
BE = BR * N_COLS
NB = N_ROWS // BR
GP = BE // 208
BIAS = 4096


def _sc_body(x_hbm, a_hbm, b_hbm, cb_hbm, yt_hbm, dy_hbm, o_hbm,
             yt_v, dy_v, a_v, b_v, cb_v):
    pltpu.sync_copy(yt_hbm, yt_v)
    pltpu.sync_copy(dy_hbm, dy_v)
    pltpu.sync_copy(a_hbm, a_v)
    pltpu.sync_copy(b_hbm, b_v)
    pltpu.sync_copy(cb_hbm, cb_v)

    def block(in_v, out_v):
        for v in range(13):
            av = a_v[pl.ds(v * 16, 16)]
            bv = b_v[pl.ds(v * 16, 16)]
            cb = cb_v[pl.ds(v * 16, 16)]

            @plsc.parallel_loop(0, GP, 1, unroll=5)
            def _(g, av=av, bv=bv, cb=cb, v=v):
                off = g * 208 + v * 16
                x = in_v[0, pl.ds(off, 16)]
                fid = x * av + bv
                ti = fid.astype(jnp.int32)
                frac = fid - ti.astype(jnp.float32)
                gi = jnp.minimum(jnp.maximum(ti, BIAS), BIAS + NS - 2) + cb
                y0 = plsc.load_gather(yt_v, [gi])
                dy = plsc.load_gather(dy_v, [gi])
                out_v[0, pl.ds(off, 16)] = y0 + frac * dy

    pltpu.emit_pipeline(
        block,
        grid=(NB,),
        in_specs=[pl.BlockSpec((1, BE), lambda i: (i, 0))],
        out_specs=[pl.BlockSpec((1, BE), lambda i: (i, 0))],
        core_axis_name=("c", "s"),
        dimension_semantics=(pltpu.PARALLEL,),
    )(x_hbm, o_hbm)


def kernel(X, transforms_X, transforms_Y):
    xb0 = transforms_X[:, 0]
    xb1 = transforms_X[:, 1]
    rinv = 1.0 / (xb1 - xb0)
    a = rinv * (NS - 1)
    b = -xb0 * rinv * (NS - 1) + BIAS
    col = jnp.arange(208, dtype=jnp.int32) % N_COLS
    a_pat = a[col]
    b_pat = b[col]
    cb_pat = col * NS - BIAS
    ytab = transforms_Y.reshape(-1)
    dtab = jnp.pad(transforms_Y[:, 1:] - transforms_Y[:, :-1],
                   ((0, 0), (0, 1))).reshape(-1)
    mesh = plsc.VectorSubcoreMesh(core_axis_name="c", subcore_axis_name="s")
    cp = pltpu.CompilerParams()
    if "needs_layout_passes" in pltpu.CompilerParams.__dataclass_fields__:
        cp = dataclasses.replace(cp, needs_layout_passes=False)

    run = functools.partial(
        pl.kernel,
        mesh=mesh,
        compiler_params=cp,
        out_type=jax.ShapeDtypeStruct((NB, BE), jnp.float32),
        scratch_types=[
            pltpu.VMEM((N_COLS * NS,), jnp.float32),
            pltpu.VMEM((N_COLS * NS,), jnp.float32),
            pltpu.VMEM((208,), jnp.float32),
            pltpu.VMEM((208,), jnp.float32),
            pltpu.VMEM((208,), jnp.int32),
        ],
    )(_sc_body)

    out = run(X.reshape(NB, BE), a_pat, b_pat, cb_pat, ytab, dtab)
    return out.reshape(N_ROWS, N_COLS)


# speedup vs baseline: 524.2539x; 1.0414x over previous
import dataclasses
import functools

import jax
import jax.numpy as jnp
from jax.experimental import pallas as pl
from jax.experimental.pallas import tpu as pltpu
from jax.experimental.pallas import tpu_sc as plsc

N_ROWS = 1000000
N_COLS = 26
NS = 200
BR = 200
NB = N_ROWS // BR
BIAS = 4096
OFF2 = N_COLS - 16


def _sc_body(x_hbm, a_hbm, b_hbm, cb_hbm, yt_hbm, dy_hbm, o_hbm,
             yt_v, dy_v, a_v, b_v, cb_v):
    pltpu.sync_copy(yt_hbm, yt_v)
    pltpu.sync_copy(dy_hbm, dy_v)
    pltpu.sync_copy(a_hbm, a_v)
    pltpu.sync_copy(b_hbm, b_v)
    pltpu.sync_copy(cb_hbm, cb_v)

    def block(in_v, out_v):
        pats = [(a_v[pl.ds(v * 16, 16)], b_v[pl.ds(v * 16, 16)],
                 cb_v[pl.ds(v * 16, 16)]) for v in range(2)]

        @plsc.parallel_loop(0, BR, 1, unroll=4)
        def _(r):
            for v, (av, bv, cb) in enumerate(pats):
                off = v * OFF2
                x = in_v[r, pl.ds(off, 16)]
                fid = x * av + bv
                ti = fid.astype(jnp.int32)
                frac = fid - ti.astype(jnp.float32)
                gi = jnp.minimum(jnp.maximum(ti, BIAS), BIAS + NS - 2) + cb
                y0 = plsc.load_gather(yt_v, [gi])
                dy = plsc.load_gather(dy_v, [gi])
                out_v[r, pl.ds(off, 16)] = y0 + frac * dy

    pltpu.emit_pipeline(
        block,
        grid=(NB,),
        in_specs=[pl.BlockSpec((BR, N_COLS), lambda i: (i, 0))],
        out_specs=[pl.BlockSpec((BR, N_COLS), lambda i: (i, 0))],
        core_axis_name=("c", "s"),
        dimension_semantics=(pltpu.PARALLEL,),
    )(x_hbm, o_hbm)


def kernel(X, transforms_X, transforms_Y):
    xb0 = transforms_X[:, 0]
    xb1 = transforms_X[:, 1]
    rinv = 1.0 / (xb1 - xb0)
    a = rinv * (NS - 1)
    b = -xb0 * rinv * (NS - 1) + BIAS
    col = jnp.concatenate([jnp.arange(16, dtype=jnp.int32),
                           jnp.arange(OFF2, N_COLS, dtype=jnp.int32)])
    a_pat = a[col]
    b_pat = b[col]
    cb_pat = col * NS - BIAS
    ytab = transforms_Y.reshape(-1)
    dtab = jnp.pad(transforms_Y[:, 1:] - transforms_Y[:, :-1],
                   ((0, 0), (0, 1))).reshape(-1)
    mesh = plsc.VectorSubcoreMesh(core_axis_name="c", subcore_axis_name="s")
    cp = pltpu.CompilerParams()
    if "needs_layout_passes" in pltpu.CompilerParams.__dataclass_fields__:
        cp = dataclasses.replace(cp, needs_layout_passes=False)

    run = functools.partial(
        pl.kernel,
        mesh=mesh,
        compiler_params=cp,
        out_type=jax.ShapeDtypeStruct((N_ROWS, N_COLS), jnp.float32),
        scratch_types=[
            pltpu.VMEM((N_COLS * NS,), jnp.float32),
            pltpu.VMEM((N_COLS * NS,), jnp.float32),
            pltpu.VMEM((32,), jnp.float32),
            pltpu.VMEM((32,), jnp.float32),
            pltpu.VMEM((32,), jnp.int32),
        ],
    )(_sc_body)

    return run(X, a_pat, b_pat, cb_pat, ytab, dtab)
